# baseline (speedup 1.0000x reference)
import functools

import jax
import jax.numpy as jnp
from jax import lax
from jax.experimental import pallas as pl
from jax.experimental.pallas import tpu as pltpu
from jax.experimental.pallas import tpu_sc as plsc

D = 64
SCALE = float(D) ** 0.5

R = 4096
T = 200
NC = 2
NS = 16
NW = NC * NS
RW = R // NW
V2 = 500000

_mesh = plsc.VectorSubcoreMesh(core_axis_name="c", subcore_axis_name="s")


@functools.partial(
    pl.kernel,
    mesh=_mesh,
    out_type=jax.ShapeDtypeStruct((T, D, R), jnp.float32),
    scratch_types=[
        pltpu.VMEM((T, RW), jnp.int32),
        pltpu.VMEM((T, RW), jnp.int32),
        pltpu.VMEM((2, RW, 128), jnp.float32),
        pltpu.VMEM((2, D, RW), jnp.float32),
        pltpu.SemaphoreType.DMA((2,)),
        pltpu.SemaphoreType.DMA((2,)),
    ],
    compiler_params=pltpu.CompilerParams(
        use_tc_tiling_on_sc=True, needs_layout_passes=False
    ),
)
def _embed(idxt_hbm, table2_hbm, out_hbm, idx_v, idx2_v, rows_v, ot_v, gsem, wsem):
    wid = lax.axis_index("s") * NC + lax.axis_index("c")
    r0 = wid * RW

    pltpu.sync_copy(idxt_hbm.at[:, pl.ds(r0, RW)], idx_v)

    def halve(t, carry):
        for g in range(RW // 16):
            sl = pl.ds(g * 16, 16)
            idx2_v[t, sl] = lax.shift_right_logical(idx_v[t, sl], 1)
        return carry

    lax.fori_loop(0, T, halve, 0, unroll=4)

    def fire_gather(t, b):
        pltpu.async_copy(table2_hbm.at[idx2_v.at[t]], rows_v.at[b], gsem.at[b])

    def wait_gather(t, b):
        pltpu.make_async_copy(
            table2_hbm.at[idx2_v.at[t]], rows_v.at[b], gsem.at[b]
        ).wait()

    def fire_write(t, b):
        pltpu.async_copy(ot_v.at[b], out_hbm.at[t, :, pl.ds(r0, RW)], wsem.at[b])

    def wait_write(t, b):
        pltpu.make_async_copy(
            ot_v.at[b], out_hbm.at[t, :, pl.ds(r0, RW)], wsem.at[b]
        ).wait()

    dvec = [lax.iota(jnp.int32, 16) + c * 16 for c in range(D // 16)]

    def compute(t, b):
        for jg in range(RW // 16):
            sl = pl.ds(jg * 16, 16)
            idx16 = idx_v[t, sl]
            s = jnp.where(idx16 != 0, SCALE, 0.0).astype(jnp.float32)
            par = (idx16 & 1) * D
            for j in range(16):
                sj = s[j]
                base = par[j]
                jv = jnp.full((16,), jg * 16 + j, jnp.int32)
                row = jg * 16 + j
                for c in range(D // 16):
                    vals = rows_v[b, row, pl.ds(base + c * 16, 16)]
                    plsc.store_scatter(ot_v.at[b], [dvec[c], jv], vals * sj)

    def step(k, carry):
        for u in range(2):
            t = k * 2 + u
            b = u
            wait_gather(t, b)

            @pl.when(t >= 2)
            def _():
                wait_write(t - 2, b)

            compute(t, b)
            fire_write(t, b)

            @pl.when(t + 2 < T)
            def _():
                fire_gather(t + 2, b)

        return carry

    fire_gather(0, 0)
    fire_gather(1, 1)
    lax.fori_loop(0, T // 2, step, 0)
    wait_write(T - 2, 0)
    wait_write(T - 1, 1)


def kernel(input, lookup_table):
    idxt = input.astype(jnp.int32).T
    table2 = lookup_table.reshape(V2, 2 * D)
    out = _embed(idxt, table2)
    return out.transpose(2, 0, 1)


# speedup vs baseline: 1.4237x; 1.4237x over previous
import functools

import jax
import jax.numpy as jnp
from jax import lax
from jax.experimental import pallas as pl
from jax.experimental.pallas import tpu as pltpu
from jax.experimental.pallas import tpu_sc as plsc

D = 64
SCALE = float(D) ** 0.5

R = 4096
T = 200
NC = 2
NS = 16
NW = NC * NS
RW = R // NW
V2 = 500000

_mesh = plsc.VectorSubcoreMesh(core_axis_name="c", subcore_axis_name="s")


@functools.partial(
    pl.kernel,
    mesh=_mesh,
    out_type=jax.ShapeDtypeStruct((T, D, R), jnp.float32),
    scratch_types=[
        pltpu.VMEM((T, RW), jnp.int32),
        pltpu.VMEM((T, RW), jnp.int32),
        pltpu.VMEM((2, RW, 128), jnp.float32),
        pltpu.VMEM((2, D, RW), jnp.float32),
        pltpu.SemaphoreType.DMA((2,)),
        pltpu.SemaphoreType.DMA((2,)),
    ],
    compiler_params=pltpu.CompilerParams(
        use_tc_tiling_on_sc=True, needs_layout_passes=False
    ),
)
def _embed(idxt_hbm, table2_hbm, out_hbm, idx_v, idx2_v, rows_v, ot_v, gsem, wsem):
    wid = lax.axis_index("s") * NC + lax.axis_index("c")
    r0 = wid * RW

    pltpu.sync_copy(idxt_hbm.at[:, pl.ds(r0, RW)], idx_v)

    def halve(t, carry):
        for g in range(RW // 16):
            sl = pl.ds(g * 16, 16)
            idx2_v[t, sl] = lax.shift_right_logical(idx_v[t, sl], 1)
        return carry

    lax.fori_loop(0, T, halve, 0, unroll=4)

    def fire_gather(t, b):
        pltpu.async_copy(table2_hbm.at[idx2_v.at[t]], rows_v.at[b], gsem.at[b])

    def wait_gather(t, b):
        pltpu.make_async_copy(
            table2_hbm.at[idx2_v.at[t]], rows_v.at[b], gsem.at[b]
        ).wait()

    def fire_write(t, b):
        pltpu.async_copy(ot_v.at[b], out_hbm.at[t, :, pl.ds(r0, RW)], wsem.at[b])

    def wait_write(t, b):
        pltpu.make_async_copy(
            ot_v.at[b], out_hbm.at[t, :, pl.ds(r0, RW)], wsem.at[b]
        ).wait()

    iota = lax.iota(jnp.int32, 16)
    civ = [iota + c * 16 for c in range(D // 16)]

    def take16(x, rot):
        return lax.gather(
            x,
            rot[:, None],
            dimension_numbers=lax.GatherDimensionNumbers(
                offset_dims=(), collapsed_slice_dims=(0,), start_index_map=(0,)
            ),
            slice_sizes=(1,),
            mode=lax.GatherScatterMode.PROMISE_IN_BOUNDS,
        )

    def compute(t, b):
        for jg in range(RW // 16):
            sl = pl.ds(jg * 16, 16)
            idx16 = idx_v[t, sl]
            s = jnp.where(idx16 != 0, SCALE, 0.0).astype(jnp.float32)
            par = (idx16 & 1) * D

            def kbody(k, carry):
                rot = (iota + k) & 15
                rows = rot + jg * 16
                s_k = take16(s, rot)
                p_k = take16(par, rot)
                for c in range(D // 16):
                    colv = p_k + civ[c]
                    vals = plsc.load_gather(rows_v.at[b], [rows, colv])
                    plsc.store_scatter(ot_v.at[b], [civ[c], rows], vals * s_k)
                return carry

            lax.fori_loop(0, 16, kbody, 0, unroll=2)

    def step(k, carry):
        for u in range(2):
            t = k * 2 + u
            b = u
            wait_gather(t, b)

            @pl.when(t >= 2)
            def _():
                wait_write(t - 2, b)

            compute(t, b)
            fire_write(t, b)

            @pl.when(t + 2 < T)
            def _():
                fire_gather(t + 2, b)

        return carry

    fire_gather(0, 0)
    fire_gather(1, 1)
    lax.fori_loop(0, T // 2, step, 0)
    wait_write(T - 2, 0)
    wait_write(T - 1, 1)


def kernel(input, lookup_table):
    idxt = input.astype(jnp.int32).T
    table2 = lookup_table.reshape(V2, 2 * D)
    out = _embed(idxt, table2)
    return out.transpose(2, 0, 1)
